# baseline (speedup 1.0000x reference)
import functools

import jax
import jax.numpy as jnp
from jax import lax
from jax.experimental import pallas as pl
from jax.experimental.pallas import tpu as pltpu
from jax.experimental.pallas import tpu_sc as plsc

_NUM_ITEMS = 1_000_000
_STAGE = 65_536
_TABLE_HBM = _STAGE * 16
_ZERO_SLOT = _NUM_ITEMS - 1
_NC, _NS = 2, 16
_NW = _NC * _NS
_BATCH, _HIST = 16384, 200
_B_TOTAL = _BATCH * _HIST
_PER_W = _B_TOTAL // _NW
_CHUNK = 12_800
_HALF = _CHUNK // 2
_N_CHUNKS = _PER_W // _CHUNK


def _remap_chunk(idx_ref):
    @plsc.parallel_loop(0, _CHUNK, step=16, unroll=8)
    def _r(off):
        iv = idx_ref[pl.ds(off, 16)]
        idx_ref[pl.ds(off, 16)] = jnp.where(
            iv == 0, jnp.int32(_ZERO_SLOT), iv - 1)


def _sigmoid_chunk(val_ref):
    @plsc.parallel_loop(0, _CHUNK, step=16, unroll=8)
    def _s(off):
        v = val_ref[pl.ds(off, 16)]
        val_ref[pl.ds(off, 16)] = 1.0 / (1.0 + jnp.exp(-v))


@functools.partial(
    pl.kernel,
    out_type=jax.ShapeDtypeStruct((_B_TOTAL,), jnp.float32),
    mesh=plsc.VectorSubcoreMesh(core_axis_name="c", subcore_axis_name="s"),
    scratch_types=[
        pltpu.VMEM((_CHUNK,), jnp.int32),
        pltpu.VMEM((_CHUNK,), jnp.int32),
        pltpu.VMEM((_CHUNK,), jnp.float32),
        pltpu.VMEM((_CHUNK,), jnp.float32),
        pltpu.VMEM_SHARED((_TABLE_HBM,), jnp.float32),
        pltpu.SemaphoreType.DMA,
        pltpu.SemaphoreType.DMA,
        pltpu.SemaphoreType.DMA,
    ],
)
def _gather_kernel(table_hbm, idx_hbm, out_hbm,
                   idx_v0, idx_v1, val_v0, val_v1, table_sp,
                   lsem, gsem, ssem):
    sid = lax.axis_index("s")
    wid = sid * _NC + lax.axis_index("c")
    base = wid * _PER_W

    soff = sid * _STAGE

    idx_bufs = [idx_v0, idx_v1]
    val_bufs = [val_v0, val_v1]
    loads = [None] * _N_CHUNKS
    loads[0] = pltpu.async_copy(idx_hbm.at[pl.ds(base, _CHUNK)], idx_v0, lsem)

    loads[0].wait()
    _remap_chunk(idx_v0)

    pltpu.sync_copy(table_hbm.at[pl.ds(soff, _STAGE)],
                    table_sp.at[pl.ds(soff, _STAGE)])

    plsc.subcore_barrier()

    stores = [None, None]
    gh = [None, None]
    for i in range(_N_CHUNKS):
        cur = i % 2
        oth = 1 - cur
        if stores[cur] is not None:
            stores[cur].wait()
        gh[cur] = pltpu.async_copy(table_sp.at[idx_bufs[cur]], val_bufs[cur],
                                   gsem)
        if i > 0:
            gh[oth].wait()
            _sigmoid_chunk(val_bufs[oth])
            stores[oth] = pltpu.async_copy(
                val_bufs[oth],
                out_hbm.at[pl.ds(base + (i - 1) * _CHUNK, _CHUNK)], ssem)
        if i + 1 < _N_CHUNKS:
            loads[i + 1] = pltpu.async_copy(
                idx_hbm.at[pl.ds(base + (i + 1) * _CHUNK, _CHUNK)],
                idx_bufs[oth], lsem)
            loads[i + 1].wait()
            _remap_chunk(idx_bufs[oth])
    last = (_N_CHUNKS - 1) % 2
    gh[last].wait()
    _sigmoid_chunk(val_bufs[last])
    stores[last] = pltpu.async_copy(
        val_bufs[last],
        out_hbm.at[pl.ds(base + (_N_CHUNKS - 1) * _CHUNK, _CHUNK)], ssem)
    for h in stores:
        if h is not None:
            h.wait()


def kernel(indices, item_rating_logits):
    idx = indices.astype(jnp.int32).reshape(-1)
    logits = item_rating_logits.astype(jnp.float32).reshape(-1)
    logits = jnp.pad(logits, (0, _TABLE_HBM - logits.shape[0]),
                     constant_values=-1e30)
    return _gather_kernel(logits, idx).reshape(_BATCH, _HIST)
